# baseline (speedup 1.0000x reference)
import jax
import jax.numpy as jnp
from jax import lax
from jax.experimental import pallas as pl
from jax.experimental.pallas import tpu as pltpu
from jax.experimental.pallas import tpu_sc as plsc

NUM_SEG = 10000
D = 128
ROWS = 320000
NC = 2
NS = 16
NW = NC * NS
K = 128
TOTCHUNK = ROWS // K
NCH = TOTCHUNK // NW + 1
NEXTRA = TOTCHUNK - NW * (NCH - 1)
ACC_ROWS = NUM_SEG + 8
WB = 624
WB_LAST = 640
ZR = 8


def _sc_scatter_body(feat_hbm, idx_hbm, out0_hbm, out1_hbm,
                     fb, ib, zbuf, acc, lsem0, lsem1, lsem2, ssem0, ssem1, zsem):
    cid = lax.axis_index("c")
    sid = lax.axis_index("s")
    w = cid * NS + sid
    lsem = (lsem0, lsem1, lsem2)
    ssem = (ssem0, ssem1)
    start_w = (78 * w + jnp.minimum(w, NEXTRA)) * K
    real78 = w < NEXTRA

    def chunk_row(i):
        if isinstance(i, int) and i < NCH - 1:
            return start_w + i * K
        return start_w + lax.select(real78, jnp.int32((NCH - 1) * K), jnp.int32(0))

    def loads(i, b, start):
        r0 = chunk_row(i) if (isinstance(i, int) and i == NCH - 1) else start_w + i * K
        ops = [pltpu.make_async_copy(feat_hbm.at[pl.ds(r0, K)], fb.at[b], lsem[b]),
               pltpu.make_async_copy(idx_hbm.at[pl.ds(r0, K)], ib.at[b], lsem[b])]
        for op in ops:
            op.start() if start else op.wait()

    def scatter(b, p, start):
        op = pltpu.make_async_copy(fb.at[b], acc.at[ib.at[b]], ssem[p])
        op.start(add=True) if start else op.wait()

    loads(0, 0, True)
    loads(1, 1, True)

    def zrow(r, carry):
        def zcol(c, carry2):
            zbuf[r, pl.ds(c * 16, 16)] = jnp.zeros((16,), jnp.float32)
            return carry2
        return lax.fori_loop(0, D // 16, zcol, carry)
    lax.fori_loop(0, ZR, zrow, 0)

    lo = sid * WB
    nzero = lax.select(sid == NS - 1, WB_LAST // ZR, WB // ZR)

    def zfire(t, carry):
        pltpu.make_async_copy(zbuf, acc.at[pl.ds(lo + t * ZR, ZR)], zsem).start()
        return carry
    lax.fori_loop(0, nzero, zfire, 0)

    def zdrain(t, carry):
        pltpu.make_async_copy(zbuf, acc.at[pl.ds(lo + t * ZR, ZR)], zsem).wait()
        return carry
    lax.fori_loop(0, nzero, zdrain, 0)
    plsc.subcore_barrier()

    def step(i, b, p, drain_prev=True, issue_next=True):
        loads(i, b, False)
        if drain_prev:
            scatter((b + 2) % 3, 1 - p, False)
        scatter(b, p, True)
        if issue_next:
            loads(i + 2, (b + 2) % 3, True)

    step(0, 0, 0, drain_prev=False)
    step(1, 1, 1)
    step(2, 2, 0)

    def body(j, carry):
        base = 3 + 6 * j
        for t in range(6):
            step(base + t, t % 3, (1 + t) % 2)
        return carry
    lax.fori_loop(0, 12, body, 0)

    step(75, 0, 1)
    step(76, 1, 0)
    step(77, 2, 1, issue_next=False)

    loads(NCH - 1, 0, False)
    scatter(2, 1, False)

    @pl.when(jnp.logical_not(real78))
    def _():
        def trash(c, carry):
            ib[0, pl.ds(c * 16, 16)] = jnp.full((16,), NUM_SEG, jnp.int32)
            return carry
        lax.fori_loop(0, K // 16, trash, 0)
    scatter(0, 0, True)
    scatter(0, 0, False)
    plsc.subcore_barrier()

    for c, out_hbm in ((0, out0_hbm), (1, out1_hbm)):
        @pl.when(jnp.logical_and(cid == c, sid < NS - 1))
        def _(out_hbm=out_hbm):
            pltpu.sync_copy(acc.at[pl.ds(lo, WB)], out_hbm.at[pl.ds(lo, WB)])

        @pl.when(jnp.logical_and(cid == c, sid == NS - 1))
        def _(out_hbm=out_hbm):
            pltpu.sync_copy(acc.at[pl.ds(lo, WB_LAST)],
                            out_hbm.at[pl.ds(lo, WB_LAST)])


_sc_scatter = pl.kernel(
    _sc_scatter_body,
    out_type=[jax.ShapeDtypeStruct((NUM_SEG, D), jnp.float32),
              jax.ShapeDtypeStruct((NUM_SEG, D), jnp.float32)],
    mesh=plsc.VectorSubcoreMesh(core_axis_name="c", subcore_axis_name="s"),
    scratch_types=[
        pltpu.VMEM((3, K, D), jnp.float32),
        pltpu.VMEM((3, K), jnp.int32),
        pltpu.VMEM((ZR, D), jnp.float32),
        pltpu.VMEM_SHARED((ACC_ROWS, D), jnp.float32),
        pltpu.SemaphoreType.DMA,
        pltpu.SemaphoreType.DMA,
        pltpu.SemaphoreType.DMA,
        pltpu.SemaphoreType.DMA,
        pltpu.SemaphoreType.DMA,
        pltpu.SemaphoreType.DMA,
    ],
)


def _combine_body(a_ref, b_ref, o_ref):
    o_ref[...] = a_ref[...] + b_ref[...]


def _tc_combine(a, b):
    blk = NUM_SEG // 10
    return pl.pallas_call(
        _combine_body,
        grid=(10,),
        in_specs=[pl.BlockSpec((blk, D), lambda i: (i, 0)),
                  pl.BlockSpec((blk, D), lambda i: (i, 0))],
        out_specs=pl.BlockSpec((blk, D), lambda i: (i, 0)),
        out_shape=jax.ShapeDtypeStruct((NUM_SEG, D), jnp.float32),
    )(a, b)


def kernel(feat, unq_inv, mode):
    del mode
    idx = unq_inv.astype(jnp.int32)
    p0, p1 = _sc_scatter(feat, idx)
    return _tc_combine(p0, p1)


# speedup vs baseline: 1.0207x; 1.0207x over previous
import jax
import jax.numpy as jnp
from jax import lax
from jax.experimental import pallas as pl
from jax.experimental.pallas import tpu as pltpu
from jax.experimental.pallas import tpu_sc as plsc

NUM_SEG = 10000
D = 128
ROWS = 320000
NC = 2
NS = 16
NW = NC * NS
K = 128
TOTCHUNK = ROWS // K
NCH = TOTCHUNK // NW + 1
NEXTRA = TOTCHUNK - NW * (NCH - 1)
ACC_ROWS = NUM_SEG + 8
WB = 624
WB_LAST = 640
ZR = 8


def _sc_scatter_body(feat_hbm, idx_hbm, out0_hbm, out1_hbm,
                     fb, ib, zbuf, acc, lsem0, lsem1, lsem2, ssem0, ssem1, zsem):
    cid = lax.axis_index("c")
    sid = lax.axis_index("s")
    w = cid * NS + sid
    lsem = (lsem0, lsem1, lsem2)
    ssem = (ssem0, ssem1)
    start_w = (78 * w + jnp.minimum(w, NEXTRA)) * K
    real78 = w < NEXTRA

    def chunk_row(i):
        if isinstance(i, int) and i < NCH - 1:
            return start_w + i * K
        return start_w + lax.select(real78, jnp.int32((NCH - 1) * K), jnp.int32(0))

    def loads(i, b, start):
        r0 = chunk_row(i) if (isinstance(i, int) and i == NCH - 1) else start_w + i * K
        ops = [pltpu.make_async_copy(feat_hbm.at[pl.ds(r0, K)], fb.at[b], lsem[b]),
               pltpu.make_async_copy(idx_hbm.at[pl.ds(r0, K)], ib.at[b], lsem[b])]
        for op in ops:
            op.start() if start else op.wait()

    def scatter(b, p, start):
        op = pltpu.make_async_copy(fb.at[b], acc.at[ib.at[b]], ssem[p])
        op.start(add=True) if start else op.wait()

    loads(0, 0, True)
    loads(1, 1, True)

    def zrow(r, carry):
        def zcol(c, carry2):
            zbuf[r, pl.ds(c * 16, 16)] = jnp.zeros((16,), jnp.float32)
            return carry2
        return lax.fori_loop(0, D // 16, zcol, carry)
    lax.fori_loop(0, ZR, zrow, 0)

    lo = sid * WB
    nzero = lax.select(sid == NS - 1, WB_LAST // ZR, WB // ZR)

    def zfire(t, carry):
        pltpu.make_async_copy(zbuf, acc.at[pl.ds(lo + t * ZR, ZR)], zsem).start()
        return carry
    lax.fori_loop(0, nzero, zfire, 0)

    def zdrain(t, carry):
        pltpu.make_async_copy(zbuf, acc.at[pl.ds(lo + t * ZR, ZR)], zsem).wait()
        return carry
    lax.fori_loop(0, nzero, zdrain, 0)
    plsc.subcore_barrier()

    def step(i, b, p, drain_prev=True, issue_next=True):
        loads(i, b, False)
        if drain_prev:
            scatter((b + 2) % 3, 1 - p, False)
        scatter(b, p, True)
        if issue_next:
            loads(i + 2, (b + 2) % 3, True)

    step(0, 0, 0, drain_prev=False)
    step(1, 1, 1)
    step(2, 2, 0)

    def body(j, carry):
        base = 3 + 6 * j
        for t in range(6):
            step(base + t, t % 3, (1 + t) % 2)
        return carry
    lax.fori_loop(0, 12, body, 0)

    step(75, 0, 1)
    step(76, 1, 0)
    step(77, 2, 1, issue_next=False)

    loads(NCH - 1, 0, False)
    scatter(2, 1, False)

    @pl.when(jnp.logical_not(real78))
    def _():
        def trash(c, carry):
            ib[0, pl.ds(c * 16, 16)] = jnp.full((16,), NUM_SEG, jnp.int32)
            return carry
        lax.fori_loop(0, K // 16, trash, 0)
    scatter(0, 0, True)
    scatter(0, 0, False)
    plsc.subcore_barrier()

    for c, out_hbm in ((0, out0_hbm), (1, out1_hbm)):
        @pl.when(jnp.logical_and(cid == c, sid < NS - 1))
        def _(out_hbm=out_hbm):
            pltpu.sync_copy(acc.at[pl.ds(lo, WB)], out_hbm.at[pl.ds(lo, WB)])

        @pl.when(jnp.logical_and(cid == c, sid == NS - 1))
        def _(out_hbm=out_hbm):
            pltpu.sync_copy(acc.at[pl.ds(lo, WB_LAST)],
                            out_hbm.at[pl.ds(lo, WB_LAST)])


_sc_scatter = pl.kernel(
    _sc_scatter_body,
    out_type=[jax.ShapeDtypeStruct((NUM_SEG, D), jnp.float32),
              jax.ShapeDtypeStruct((NUM_SEG, D), jnp.float32)],
    mesh=plsc.VectorSubcoreMesh(core_axis_name="c", subcore_axis_name="s"),
    scratch_types=[
        pltpu.VMEM((3, K, D), jnp.float32),
        pltpu.VMEM((3, K), jnp.int32),
        pltpu.VMEM((ZR, D), jnp.float32),
        pltpu.VMEM_SHARED((ACC_ROWS, D), jnp.float32),
        pltpu.SemaphoreType.DMA,
        pltpu.SemaphoreType.DMA,
        pltpu.SemaphoreType.DMA,
        pltpu.SemaphoreType.DMA,
        pltpu.SemaphoreType.DMA,
        pltpu.SemaphoreType.DMA,
    ],
)


def _combine_body(a_ref, b_ref, o_ref):
    o_ref[...] = a_ref[...] + b_ref[...]


def _tc_combine(a, b):
    blk = NUM_SEG // 5
    return pl.pallas_call(
        _combine_body,
        grid=(5,),
        in_specs=[pl.BlockSpec((blk, D), lambda i: (i, 0)),
                  pl.BlockSpec((blk, D), lambda i: (i, 0))],
        out_specs=pl.BlockSpec((blk, D), lambda i: (i, 0)),
        out_shape=jax.ShapeDtypeStruct((NUM_SEG, D), jnp.float32),
    )(a, b)


def kernel(feat, unq_inv, mode):
    del mode
    idx = unq_inv.astype(jnp.int32)
    p0, p1 = _sc_scatter(feat, idx)
    return _tc_combine(p0, p1)


# speedup vs baseline: 1.0296x; 1.0088x over previous
import jax
import jax.numpy as jnp
from jax import lax
from jax.experimental import pallas as pl
from jax.experimental.pallas import tpu as pltpu
from jax.experimental.pallas import tpu_sc as plsc

NUM_SEG = 10000
D = 128
ROWS = 320000
NC = 2
NS = 16
NW = NC * NS
K = 80
ROWS_PER_TILE = ROWS // NW
NCH = ROWS_PER_TILE // K
WB = 624
WB_LAST = 640
ZR = 8


def _sc_scatter_body(feat_hbm, idx_hbm, out0_hbm, out1_hbm,
                     fb, ib, zbuf, acc,
                     lsem0, lsem1, lsem2, lsem3, ssem0, ssem1, zsem):
    cid = lax.axis_index("c")
    sid = lax.axis_index("s")
    w = cid * NS + sid
    lsem = (lsem0, lsem1, lsem2, lsem3)
    ssem = (ssem0, ssem1)
    start_w = w * ROWS_PER_TILE

    def loads(i, b, start):
        r0 = start_w + i * K
        ops = [pltpu.make_async_copy(feat_hbm.at[pl.ds(r0, K)], fb.at[b], lsem[b]),
               pltpu.make_async_copy(idx_hbm.at[pl.ds(r0, K)], ib.at[b], lsem[b])]
        for op in ops:
            op.start() if start else op.wait()

    def scatter(b, p, start):
        op = pltpu.make_async_copy(fb.at[b], acc.at[ib.at[b]], ssem[p])
        op.start(add=True) if start else op.wait()

    loads(0, 0, True)
    loads(1, 1, True)
    loads(2, 2, True)

    def zrow(r, carry):
        def zcol(c, carry2):
            zbuf[r, pl.ds(c * 16, 16)] = jnp.zeros((16,), jnp.float32)
            return carry2
        return lax.fori_loop(0, D // 16, zcol, carry)
    lax.fori_loop(0, ZR, zrow, 0)

    lo = sid * WB
    nzero = lax.select(sid == NS - 1, WB_LAST // ZR, WB // ZR)

    def zfire(t, carry):
        pltpu.make_async_copy(zbuf, acc.at[pl.ds(lo + t * ZR, ZR)], zsem).start()
        return carry
    lax.fori_loop(0, nzero, zfire, 0)

    def zdrain(t, carry):
        pltpu.make_async_copy(zbuf, acc.at[pl.ds(lo + t * ZR, ZR)], zsem).wait()
        return carry
    lax.fori_loop(0, nzero, zdrain, 0)
    plsc.subcore_barrier()

    def step(i, b, p, drain_prev=True, issue_next=True):
        loads(i, b, False)
        if drain_prev:
            scatter((b + 3) % 4, 1 - p, False)
        scatter(b, p, True)
        if issue_next:
            loads(i + 3, (b + 3) % 4, True)

    step(0, 0, 0, drain_prev=False)
    step(1, 1, 1)
    step(2, 2, 0)
    step(3, 3, 1)

    def body(j, carry):
        base = 4 * j
        for t in range(4):
            step(base + t, t, t % 2)
        return carry
    lax.fori_loop(1, 30, body, 0)

    step(120, 0, 0)
    step(121, 1, 1)
    step(122, 2, 0, issue_next=False)
    step(123, 3, 1, issue_next=False)
    step(124, 0, 0, issue_next=False)
    scatter(0, 0, False)
    plsc.subcore_barrier()

    for c, out_hbm in ((0, out0_hbm), (1, out1_hbm)):
        @pl.when(jnp.logical_and(cid == c, sid < NS - 1))
        def _(out_hbm=out_hbm):
            pltpu.sync_copy(acc.at[pl.ds(lo, WB)], out_hbm.at[pl.ds(lo, WB)])

        @pl.when(jnp.logical_and(cid == c, sid == NS - 1))
        def _(out_hbm=out_hbm):
            pltpu.sync_copy(acc.at[pl.ds(lo, WB_LAST)],
                            out_hbm.at[pl.ds(lo, WB_LAST)])


_sc_scatter = pl.kernel(
    _sc_scatter_body,
    out_type=[jax.ShapeDtypeStruct((NUM_SEG, D), jnp.float32),
              jax.ShapeDtypeStruct((NUM_SEG, D), jnp.float32)],
    mesh=plsc.VectorSubcoreMesh(core_axis_name="c", subcore_axis_name="s"),
    scratch_types=[
        pltpu.VMEM((4, K, D), jnp.float32),
        pltpu.VMEM((4, K), jnp.int32),
        pltpu.VMEM((ZR, D), jnp.float32),
        pltpu.VMEM_SHARED((NUM_SEG, D), jnp.float32),
        pltpu.SemaphoreType.DMA,
        pltpu.SemaphoreType.DMA,
        pltpu.SemaphoreType.DMA,
        pltpu.SemaphoreType.DMA,
        pltpu.SemaphoreType.DMA,
        pltpu.SemaphoreType.DMA,
        pltpu.SemaphoreType.DMA,
    ],
)


def _combine_body(a_ref, b_ref, o_ref):
    o_ref[...] = a_ref[...] + b_ref[...]


def _tc_combine(a, b):
    blk = NUM_SEG // 5
    return pl.pallas_call(
        _combine_body,
        grid=(5,),
        in_specs=[pl.BlockSpec((blk, D), lambda i: (i, 0)),
                  pl.BlockSpec((blk, D), lambda i: (i, 0))],
        out_specs=pl.BlockSpec((blk, D), lambda i: (i, 0)),
        out_shape=jax.ShapeDtypeStruct((NUM_SEG, D), jnp.float32),
    )(a, b)


def kernel(feat, unq_inv, mode):
    del mode
    idx = unq_inv.astype(jnp.int32)
    p0, p1 = _sc_scatter(feat, idx)
    return _tc_combine(p0, p1)


# speedup vs baseline: 1.0882x; 1.0569x over previous
import jax
import jax.numpy as jnp
from jax import lax
from jax.experimental import pallas as pl
from jax.experimental.pallas import tpu as pltpu
from jax.experimental.pallas import tpu_sc as plsc

NUM_SEG = 10000
D = 128
ROWS = 320000
NC = 2
NS = 16
NW = NC * NS
K = 80
ROWS_PER_TILE = ROWS // NW
NCH = ROWS_PER_TILE // K
WB = 624
WB_LAST = 640
ZR = 8


def _sc_scatter_body(feat_hbm, idx_hbm, out0_hbm, out1_hbm,
                     fb, ib, zbuf, acc,
                     lsem0, lsem1, lsem2, lsem3, ssem0, ssem1, zsem):
    cid = lax.axis_index("c")
    sid = lax.axis_index("s")
    w = cid * NS + sid
    lsem = (lsem0, lsem1, lsem2, lsem3)
    ssem = (ssem0, ssem1)
    start_w = w * ROWS_PER_TILE

    def loads(i, b, start):
        r0 = (i * NW + w) * K
        ops = [pltpu.make_async_copy(feat_hbm.at[pl.ds(r0, K)], fb.at[b], lsem[b]),
               pltpu.make_async_copy(idx_hbm.at[pl.ds(r0, K)], ib.at[b], lsem[b])]
        for op in ops:
            op.start() if start else op.wait()

    def scatter(b, p, start):
        op = pltpu.make_async_copy(fb.at[b], acc.at[ib.at[b]], ssem[p])
        op.start(add=True) if start else op.wait()

    loads(0, 0, True)
    loads(1, 1, True)
    loads(2, 2, True)

    def zrow(r, carry):
        def zcol(c, carry2):
            zbuf[r, pl.ds(c * 16, 16)] = jnp.zeros((16,), jnp.float32)
            return carry2
        return lax.fori_loop(0, D // 16, zcol, carry)
    lax.fori_loop(0, ZR, zrow, 0)

    lo = sid * WB
    nzero = lax.select(sid == NS - 1, WB_LAST // ZR, WB // ZR)

    def zfire(t, carry):
        pltpu.make_async_copy(zbuf, acc.at[pl.ds(lo + t * ZR, ZR)], zsem).start()
        return carry
    lax.fori_loop(0, nzero, zfire, 0)

    def zdrain(t, carry):
        pltpu.make_async_copy(zbuf, acc.at[pl.ds(lo + t * ZR, ZR)], zsem).wait()
        return carry
    lax.fori_loop(0, nzero, zdrain, 0)
    plsc.subcore_barrier()

    def step(i, b, p, drain_prev=True, issue_next=True):
        loads(i, b, False)
        if drain_prev:
            scatter((b + 3) % 4, 1 - p, False)
        scatter(b, p, True)
        if issue_next:
            loads(i + 3, (b + 3) % 4, True)

    step(0, 0, 0, drain_prev=False)
    step(1, 1, 1)
    step(2, 2, 0)
    step(3, 3, 1)

    def body(j, carry):
        base = 4 * j
        for t in range(4):
            step(base + t, t, t % 2)
        return carry
    lax.fori_loop(1, 30, body, 0)

    step(120, 0, 0)
    step(121, 1, 1)
    step(122, 2, 0, issue_next=False)
    step(123, 3, 1, issue_next=False)
    step(124, 0, 0, issue_next=False)
    scatter(0, 0, False)
    plsc.subcore_barrier()

    for c, out_hbm in ((0, out0_hbm), (1, out1_hbm)):
        @pl.when(jnp.logical_and(cid == c, sid < NS - 1))
        def _(out_hbm=out_hbm):
            pltpu.sync_copy(acc.at[pl.ds(lo, WB)], out_hbm.at[pl.ds(lo, WB)])

        @pl.when(jnp.logical_and(cid == c, sid == NS - 1))
        def _(out_hbm=out_hbm):
            pltpu.sync_copy(acc.at[pl.ds(lo, WB_LAST)],
                            out_hbm.at[pl.ds(lo, WB_LAST)])


_sc_scatter = pl.kernel(
    _sc_scatter_body,
    out_type=[jax.ShapeDtypeStruct((NUM_SEG, D), jnp.float32),
              jax.ShapeDtypeStruct((NUM_SEG, D), jnp.float32)],
    mesh=plsc.VectorSubcoreMesh(core_axis_name="c", subcore_axis_name="s"),
    scratch_types=[
        pltpu.VMEM((4, K, D), jnp.float32),
        pltpu.VMEM((4, K), jnp.int32),
        pltpu.VMEM((ZR, D), jnp.float32),
        pltpu.VMEM_SHARED((NUM_SEG, D), jnp.float32),
        pltpu.SemaphoreType.DMA,
        pltpu.SemaphoreType.DMA,
        pltpu.SemaphoreType.DMA,
        pltpu.SemaphoreType.DMA,
        pltpu.SemaphoreType.DMA,
        pltpu.SemaphoreType.DMA,
        pltpu.SemaphoreType.DMA,
    ],
)


def _combine_body(a_ref, b_ref, o_ref):
    o_ref[...] = a_ref[...] + b_ref[...]


def _tc_combine(a, b):
    blk = NUM_SEG // 5
    return pl.pallas_call(
        _combine_body,
        grid=(5,),
        in_specs=[pl.BlockSpec((blk, D), lambda i: (i, 0)),
                  pl.BlockSpec((blk, D), lambda i: (i, 0))],
        out_specs=pl.BlockSpec((blk, D), lambda i: (i, 0)),
        out_shape=jax.ShapeDtypeStruct((NUM_SEG, D), jnp.float32),
    )(a, b)


def kernel(feat, unq_inv, mode):
    del mode
    idx = unq_inv.astype(jnp.int32)
    p0, p1 = _sc_scatter(feat, idx)
    return _tc_combine(p0, p1)
